# Initial kernel scaffold; baseline (speedup 1.0000x reference)
import functools

import jax
import jax.numpy as jnp
from jax import lax
from jax.experimental import pallas as pl
from jax.experimental.pallas import tpu as pltpu
from jax.experimental.pallas import tpu_sc as plsc

NC = 2
NS = 16
NW = NC * NS

ROWL = 128
CHUNK_ROWS = 16
CHUNK_E = ROWL * CHUNK_ROWS


def _round_up(a, m):
    return (a + m - 1) // m * m


def _make_deg_kernel(Np, Er):
    rows_pt = Np // NS
    er_pw = Er // NW
    n_chunks = er_pw // CHUNK_ROWS
    mesh = plsc.VectorSubcoreMesh(core_axis_name="c", subcore_axis_name="s")

    @functools.partial(
        pl.kernel,
        mesh=mesh,
        out_type=jax.ShapeDtypeStruct((NC, NS, rows_pt), jnp.float32),
        scratch_types=[
            pltpu.VMEM((CHUNK_ROWS, ROWL), jnp.int32),
            pltpu.VMEM((CHUNK_E,), jnp.float32),
            pltpu.VMEM((rows_pt,), jnp.float32),
            pltpu.VMEM_SHARED((Np,), jnp.float32),
            pltpu.SemaphoreType.DMA,
        ],
    )
    def deg_kernel(dst_hbm, ew_hbm, out_hbm, dbuf, ebuf, zbuf, acc, sem):
        cid = lax.axis_index("c")
        sid = lax.axis_index("s")
        wid = cid * NS + sid

        def zero_body(i, _):
            zbuf[pl.ds(i * 16, 16)] = jnp.zeros((16,), jnp.float32)
            return 0

        lax.fori_loop(0, rows_pt // 16, zero_body, 0)
        pltpu.sync_copy(zbuf, acc.at[pl.ds(sid * rows_pt, rows_pt)])
        plsc.subcore_barrier()

        base_row = wid * er_pw

        def chunk_body(c, _):
            row0 = base_row + c * CHUNK_ROWS
            pltpu.sync_copy(dst_hbm.at[pl.ds(row0, CHUNK_ROWS)], dbuf)
            pltpu.sync_copy(ew_hbm.at[pl.ds(row0 * ROWL, CHUNK_E)], ebuf)
            descs = [
                pltpu.async_copy(
                    ebuf.at[pl.ds(j * ROWL, ROWL)], acc.at[dbuf.at[j]], sem,
                    add=True)
                for j in range(CHUNK_ROWS)
            ]
            for d in descs:
                d.wait()
            return 0

        lax.fori_loop(0, n_chunks, chunk_body, 0)
        plsc.subcore_barrier()

        pltpu.sync_copy(acc.at[pl.ds(sid * rows_pt, rows_pt)], zbuf)
        pltpu.sync_copy(zbuf, out_hbm.at[cid, sid])

    return deg_kernel


def _make_agg_kernel(Np, Er, F):
    rows_pt = Np // NS
    er_pw = Er // NW
    n_chunks = er_pw // CHUNK_ROWS
    ZR = 448
    assert rows_pt % ZR == 0
    mesh = plsc.VectorSubcoreMesh(core_axis_name="c", subcore_axis_name="s")

    @functools.partial(
        pl.kernel,
        mesh=mesh,
        out_type=jax.ShapeDtypeStruct((NC, NS, rows_pt, F), jnp.float32),
        scratch_types=[
            pltpu.VMEM((CHUNK_ROWS, ROWL), jnp.int32),
            pltpu.VMEM((CHUNK_ROWS, ROWL), jnp.int32),
            pltpu.VMEM((CHUNK_E,), jnp.float32),
            pltpu.VMEM((CHUNK_E, F), jnp.float32),
            pltpu.VMEM((448, F), jnp.float32),
            pltpu.VMEM_SHARED((Np, F), jnp.float32),
            pltpu.SemaphoreType.DMA,
            pltpu.SemaphoreType.DMA,
        ],
    )
    def agg_kernel(hw_hbm, src_hbm, dst_hbm, ew_hbm, out_hbm,
                   sbuf, dbuf, ebuf, rbuf, zbuf, acc, gsem, ssem):
        cid = lax.axis_index("c")
        sid = lax.axis_index("s")
        wid = cid * NS + sid
        ZR = 448

        def zero_body(i, _):
            for f0 in range(0, F, 16):
                zbuf[i, pl.ds(f0, 16)] = jnp.zeros((16,), jnp.float32)
            return 0

        lax.fori_loop(0, ZR, zero_body, 0)
        for t in range(rows_pt // ZR):
            pltpu.sync_copy(zbuf, acc.at[pl.ds(sid * rows_pt + t * ZR, ZR)])
        plsc.subcore_barrier()

        base_row = wid * er_pw

        def chunk_body(c, _):
            row0 = base_row + c * CHUNK_ROWS
            pltpu.sync_copy(src_hbm.at[pl.ds(row0, CHUNK_ROWS)], sbuf)
            pltpu.sync_copy(dst_hbm.at[pl.ds(row0, CHUNK_ROWS)], dbuf)
            pltpu.sync_copy(ew_hbm.at[pl.ds(row0 * ROWL, CHUNK_E)], ebuf)
            gds = [
                pltpu.async_copy(
                    hw_hbm.at[sbuf.at[j]],
                    rbuf.at[pl.ds(j * ROWL, ROWL)], gsem)
                for j in range(CHUNK_ROWS)
            ]
            for d in gds:
                d.wait()

            def mul_body(i, _):
                e0 = i * 16
                for j in range(16):
                    e = e0 + j
                    s = jnp.full((16,), ebuf[e], jnp.float32)
                    for f0 in range(0, F, 16):
                        rbuf[e, pl.ds(f0, 16)] = rbuf[e, pl.ds(f0, 16)] * s
                return 0

            lax.fori_loop(0, CHUNK_E // 16, mul_body, 0)

            sds = [
                pltpu.async_copy(
                    rbuf.at[pl.ds(j * ROWL, ROWL)],
                    acc.at[dbuf.at[j]], ssem, add=True)
                for j in range(CHUNK_ROWS)
            ]
            for d in sds:
                d.wait()
            return 0

        lax.fori_loop(0, n_chunks, chunk_body, 0)
        plsc.subcore_barrier()

        for t in range(rows_pt // ZR):
            pltpu.sync_copy(acc.at[pl.ds(sid * rows_pt + t * ZR, ZR)], zbuf)
            pltpu.sync_copy(zbuf, out_hbm.at[cid, sid, pl.ds(t * ZR, ZR)])

    return agg_kernel


def _mm_first(x, W1, degA, degB, Np, BM):
    grid = (Np // BM,)
    kd = x.shape[1]
    Fo = W1.shape[1]

    def body(x_ref, w_ref, da_ref, db_ref, hw_ref, dinv_ref):
        dinv = lax.rsqrt(1.0 + da_ref[...] + db_ref[...])
        hw_ref[...] = dinv * jnp.dot(
            x_ref[...], w_ref[...], preferred_element_type=jnp.float32)
        dinv_ref[...] = dinv

    return pl.pallas_call(
        body,
        grid=grid,
        in_specs=[
            pl.BlockSpec((BM, kd), lambda i: (i, 0)),
            pl.BlockSpec((kd, Fo), lambda i: (0, 0)),
            pl.BlockSpec((BM, 1), lambda i: (i, 0)),
            pl.BlockSpec((BM, 1), lambda i: (i, 0)),
        ],
        out_specs=[
            pl.BlockSpec((BM, Fo), lambda i: (i, 0)),
            pl.BlockSpec((BM, 1), lambda i: (i, 0)),
        ],
        out_shape=[
            jax.ShapeDtypeStruct((Np, Fo), jnp.float32),
            jax.ShapeDtypeStruct((Np, 1), jnp.float32),
        ],
    )(x, W1, degA, degB)


def _mm_mid(pa, pb, hw, dinv, b, Wn, Np, BM):
    grid = (Np // BM,)
    F = hw.shape[1]
    Fo = Wn.shape[1]

    def body(pa_ref, pb_ref, hw_ref, dinv_ref, b_ref, w_ref, out_ref):
        d = dinv_ref[...]
        h = jnp.maximum(
            d * (pa_ref[...] + pb_ref[...] + hw_ref[...]) + b_ref[...], 0.0)
        out_ref[...] = d * jnp.dot(
            h, w_ref[...], preferred_element_type=jnp.float32)

    return pl.pallas_call(
        body,
        grid=grid,
        in_specs=[
            pl.BlockSpec((BM, F), lambda i: (i, 0)),
            pl.BlockSpec((BM, F), lambda i: (i, 0)),
            pl.BlockSpec((BM, F), lambda i: (i, 0)),
            pl.BlockSpec((BM, 1), lambda i: (i, 0)),
            pl.BlockSpec((1, F), lambda i: (0, 0)),
            pl.BlockSpec((F, Fo), lambda i: (0, 0)),
        ],
        out_specs=pl.BlockSpec((BM, Fo), lambda i: (i, 0)),
        out_shape=jax.ShapeDtypeStruct((Np, Fo), jnp.float32),
    )(pa, pb, hw, dinv, b, Wn)


def _mm_final(pa, pb, hw, dinv, b3, Wfc, bfc, Np, BM):
    grid = (Np // BM,)
    F = hw.shape[1]
    Fo = Wfc.shape[1]

    def body(pa_ref, pb_ref, hw_ref, dinv_ref, b_ref, w_ref, bfc_ref, out_ref):
        d = dinv_ref[...]
        h = jnp.maximum(
            d * (pa_ref[...] + pb_ref[...] + hw_ref[...]) + b_ref[...], 0.0)
        logits = jnp.dot(
            h, w_ref[...], preferred_element_type=jnp.float32) + bfc_ref[...]
        m = jnp.max(logits, axis=1, keepdims=True)
        lse = m + jnp.log(jnp.sum(jnp.exp(logits - m), axis=1, keepdims=True))
        out_ref[...] = logits - lse

    return pl.pallas_call(
        body,
        grid=grid,
        in_specs=[
            pl.BlockSpec((BM, F), lambda i: (i, 0)),
            pl.BlockSpec((BM, F), lambda i: (i, 0)),
            pl.BlockSpec((BM, F), lambda i: (i, 0)),
            pl.BlockSpec((BM, 1), lambda i: (i, 0)),
            pl.BlockSpec((1, F), lambda i: (0, 0)),
            pl.BlockSpec((F, Fo), lambda i: (0, 0)),
            pl.BlockSpec((1, Fo), lambda i: (0, 0)),
        ],
        out_specs=pl.BlockSpec((BM, Fo), lambda i: (i, 0)),
        out_shape=jax.ShapeDtypeStruct((Np, Fo), jnp.float32),
    )(pa, pb, hw, dinv, b3, Wfc, bfc)


def kernel(x, edge_index, edge_weight, W1, b1, W2, b2, W3, b3, Wfc, bfc):
    N = x.shape[0]
    E = edge_index.shape[1]

    Np = _round_up(N, NS * 448)
    BM = Np // 16
    Ep = _round_up(E, ROWL * CHUNK_ROWS * NW)
    Er = Ep // ROWL

    xp = jnp.pad(x, ((0, Np - N), (0, 0)))
    pe = Ep - E
    src = jnp.pad(edge_index[0], (0, pe)).reshape(Er, ROWL)
    dst = jnp.pad(edge_index[1], (0, pe), constant_values=N).reshape(Er, ROWL)
    ew = jnp.pad(edge_weight, (0, pe))

    degp = _make_deg_kernel(Np, Er)(dst, ew)
    degp = degp.reshape(NC, Np, 1)

    hw1, dinv = _mm_first(xp, W1, degp[0], degp[1], Np, BM)

    agg1 = _make_agg_kernel(Np, Er, 32)(hw1, src, dst, ew)
    agg1 = agg1.reshape(NC, Np, 32)
    hw2 = _mm_mid(agg1[0], agg1[1], hw1, dinv, b1.reshape(1, -1), W2, Np, BM)

    agg2 = _make_agg_kernel(Np, Er, 16)(hw2, src, dst, ew)
    agg2 = agg2.reshape(NC, Np, 16)
    hw3 = _mm_mid(agg2[0], agg2[1], hw2, dinv, b2.reshape(1, -1), W3, Np, BM)

    agg3 = _make_agg_kernel(Np, Er, 8)(hw3, src, dst, ew)
    agg3 = agg3.reshape(NC, Np, 8)
    out = _mm_final(agg3[0], agg3[1], hw3, dinv, b3.reshape(1, -1),
                    Wfc, bfc.reshape(1, -1), Np, BM)

    return out[:N]


# speedup vs baseline: 21.9028x; 21.9028x over previous
import functools

import jax
import jax.numpy as jnp
from jax import lax
from jax.experimental import pallas as pl
from jax.experimental.pallas import tpu as pltpu
from jax.experimental.pallas import tpu_sc as plsc

NC = 2
NS = 16
NW = NC * NS

ROWL = 128
CHUNK_ROWS = 16
CHUNK_E = ROWL * CHUNK_ROWS


def _round_up(a, m):
    return (a + m - 1) // m * m


def _make_deg_kernel(Np, Er):
    rows_pt = Np // NS
    er_pw = Er // NW
    n_chunks = er_pw // CHUNK_ROWS
    mesh = plsc.VectorSubcoreMesh(core_axis_name="c", subcore_axis_name="s")

    @functools.partial(
        pl.kernel,
        mesh=mesh,
        compiler_params=pltpu.CompilerParams(use_tc_tiling_on_sc=False),
        out_type=jax.ShapeDtypeStruct((NC, NS, rows_pt), jnp.float32),
        scratch_types=[
            pltpu.VMEM((CHUNK_ROWS, ROWL), jnp.int32),
            pltpu.VMEM((CHUNK_E,), jnp.float32),
            pltpu.VMEM((rows_pt,), jnp.float32),
            pltpu.VMEM_SHARED((Np,), jnp.float32),
            pltpu.SemaphoreType.DMA,
        ],
    )
    def deg_kernel(dst_hbm, ew_hbm, out_hbm, dbuf, ebuf, zbuf, acc, sem):
        cid = lax.axis_index("c")
        sid = lax.axis_index("s")
        wid = cid * NS + sid

        def zero_body(i, _):
            zbuf[pl.ds(i * 16, 16)] = jnp.zeros((16,), jnp.float32)
            return 0

        lax.fori_loop(0, rows_pt // 16, zero_body, 0)
        pltpu.sync_copy(zbuf, acc.at[pl.ds(sid * rows_pt, rows_pt)])
        plsc.subcore_barrier()

        base_row = wid * er_pw

        def chunk_body(c, _):
            row0 = base_row + c * CHUNK_ROWS
            pltpu.sync_copy(dst_hbm.at[pl.ds(row0, CHUNK_ROWS)], dbuf)
            pltpu.sync_copy(ew_hbm.at[pl.ds(row0 * ROWL, CHUNK_E)], ebuf)
            descs = [
                pltpu.async_copy(
                    ebuf.at[pl.ds(j * ROWL, ROWL)], acc.at[dbuf.at[j]], sem,
                    add=True)
                for j in range(CHUNK_ROWS)
            ]
            for d in descs:
                d.wait()
            return 0

        lax.fori_loop(0, n_chunks, chunk_body, 0)
        plsc.subcore_barrier()

        pltpu.sync_copy(acc.at[pl.ds(sid * rows_pt, rows_pt)], zbuf)
        pltpu.sync_copy(zbuf, out_hbm.at[cid, sid])

    return deg_kernel


def _make_agg_kernel(Np, Er, F):
    rows_pt = Np // NS
    er_pw = Er // NW
    n_chunks = er_pw // CHUNK_ROWS
    ZR = 448
    assert rows_pt % ZR == 0
    mesh = plsc.VectorSubcoreMesh(core_axis_name="c", subcore_axis_name="s")

    @functools.partial(
        pl.kernel,
        mesh=mesh,
        compiler_params=pltpu.CompilerParams(use_tc_tiling_on_sc=False),
        out_type=jax.ShapeDtypeStruct((NC, NS, rows_pt, F), jnp.float32),
        scratch_types=[
            pltpu.VMEM((CHUNK_ROWS, ROWL), jnp.int32),
            pltpu.VMEM((CHUNK_ROWS, ROWL), jnp.int32),
            pltpu.VMEM((CHUNK_E,), jnp.float32),
            pltpu.VMEM((CHUNK_E, F), jnp.float32),
            pltpu.VMEM((448, F), jnp.float32),
            pltpu.VMEM_SHARED((Np, F), jnp.float32),
            pltpu.SemaphoreType.DMA,
            pltpu.SemaphoreType.DMA,
        ],
    )
    def agg_kernel(hw_hbm, src_hbm, dst_hbm, ew_hbm, out_hbm,
                   sbuf, dbuf, ebuf, rbuf, zbuf, acc, gsem, ssem):
        cid = lax.axis_index("c")
        sid = lax.axis_index("s")
        wid = cid * NS + sid
        ZR = 448

        def zero_body(i, _):
            for f0 in range(0, F, 16):
                zbuf[i, pl.ds(f0, 16)] = jnp.zeros((16,), jnp.float32)
            return 0

        lax.fori_loop(0, ZR, zero_body, 0)
        for t in range(rows_pt // ZR):
            pltpu.sync_copy(zbuf, acc.at[pl.ds(sid * rows_pt + t * ZR, ZR)])
        plsc.subcore_barrier()

        base_row = wid * er_pw

        def chunk_body(c, _):
            row0 = base_row + c * CHUNK_ROWS
            pltpu.sync_copy(src_hbm.at[pl.ds(row0, CHUNK_ROWS)], sbuf)
            pltpu.sync_copy(dst_hbm.at[pl.ds(row0, CHUNK_ROWS)], dbuf)
            pltpu.sync_copy(ew_hbm.at[pl.ds(row0 * ROWL, CHUNK_E)], ebuf)
            gds = [
                pltpu.async_copy(
                    hw_hbm.at[sbuf.at[j]],
                    rbuf.at[pl.ds(j * ROWL, ROWL)], gsem)
                for j in range(CHUNK_ROWS)
            ]
            for d in gds:
                d.wait()

            def mul_body(i, _):
                e0 = i * 16
                ewv = ebuf[pl.ds(e0, 16)]
                for j in range(16):
                    e = e0 + j
                    s = jnp.full((16,), ewv[j], jnp.float32)
                    for f0 in range(0, F, 16):
                        rbuf[e, pl.ds(f0, 16)] = rbuf[e, pl.ds(f0, 16)] * s
                return 0

            lax.fori_loop(0, CHUNK_E // 16, mul_body, 0)

            sds = [
                pltpu.async_copy(
                    rbuf.at[pl.ds(j * ROWL, ROWL)],
                    acc.at[dbuf.at[j]], ssem, add=True)
                for j in range(CHUNK_ROWS)
            ]
            for d in sds:
                d.wait()
            return 0

        lax.fori_loop(0, n_chunks, chunk_body, 0)
        plsc.subcore_barrier()

        for t in range(rows_pt // ZR):
            pltpu.sync_copy(acc.at[pl.ds(sid * rows_pt + t * ZR, ZR)], zbuf)
            pltpu.sync_copy(zbuf, out_hbm.at[cid, sid, pl.ds(t * ZR, ZR)])

    return agg_kernel


def _mm_first(x, W1, degA, degB, Np, BM):
    grid = (Np // BM,)
    kd = x.shape[1]
    Fo = W1.shape[1]
    Fh = Fo // 2

    def body(x_ref, w_ref, da_ref, db_ref, hwa_ref, hwb_ref, dinv_ref):
        dinv = 1.0 / jnp.sqrt(1.0 + da_ref[...] + db_ref[...])
        hw = dinv * jnp.dot(
            x_ref[...], w_ref[...], preferred_element_type=jnp.float32)
        hwa_ref[...] = hw[:, :Fh]
        hwb_ref[...] = hw[:, Fh:]
        dinv_ref[...] = dinv

    return pl.pallas_call(
        body,
        grid=grid,
        in_specs=[
            pl.BlockSpec((BM, kd), lambda i: (i, 0)),
            pl.BlockSpec((kd, Fo), lambda i: (0, 0)),
            pl.BlockSpec((BM, 1), lambda i: (i, 0)),
            pl.BlockSpec((BM, 1), lambda i: (i, 0)),
        ],
        out_specs=[
            pl.BlockSpec((BM, Fh), lambda i: (i, 0)),
            pl.BlockSpec((BM, Fh), lambda i: (i, 0)),
            pl.BlockSpec((BM, 1), lambda i: (i, 0)),
        ],
        out_shape=[
            jax.ShapeDtypeStruct((Np, Fh), jnp.float32),
            jax.ShapeDtypeStruct((Np, Fh), jnp.float32),
            jax.ShapeDtypeStruct((Np, 1), jnp.float32),
        ],
    )(x, W1, degA, degB)


def _mm_mid2(pa1, pb1, pa2, pb2, hwa, hwb, dinv, b, Wn, Np, BM):
    grid = (Np // BM,)
    Fh = hwa.shape[1]
    Fo = Wn.shape[1]

    def body(pa1_ref, pb1_ref, pa2_ref, pb2_ref, hwa_ref, hwb_ref,
             dinv_ref, b_ref, w_ref, out_ref):
        d = dinv_ref[...]
        agg = jnp.concatenate(
            [pa1_ref[...] + pb1_ref[...] + hwa_ref[...],
             pa2_ref[...] + pb2_ref[...] + hwb_ref[...]], axis=1)
        h = jnp.maximum(d * agg + b_ref[...], 0.0)
        out_ref[...] = d * jnp.dot(
            h, w_ref[...], preferred_element_type=jnp.float32)

    return pl.pallas_call(
        body,
        grid=grid,
        in_specs=[
            pl.BlockSpec((BM, Fh), lambda i: (i, 0)),
            pl.BlockSpec((BM, Fh), lambda i: (i, 0)),
            pl.BlockSpec((BM, Fh), lambda i: (i, 0)),
            pl.BlockSpec((BM, Fh), lambda i: (i, 0)),
            pl.BlockSpec((BM, Fh), lambda i: (i, 0)),
            pl.BlockSpec((BM, Fh), lambda i: (i, 0)),
            pl.BlockSpec((BM, 1), lambda i: (i, 0)),
            pl.BlockSpec((1, 2 * Fh), lambda i: (0, 0)),
            pl.BlockSpec((2 * Fh, Fo), lambda i: (0, 0)),
        ],
        out_specs=pl.BlockSpec((BM, Fo), lambda i: (i, 0)),
        out_shape=jax.ShapeDtypeStruct((Np, Fo), jnp.float32),
    )(pa1, pb1, pa2, pb2, hwa, hwb, dinv, b, Wn)


def _mm_mid(pa, pb, hw, dinv, b, Wn, Np, BM):
    grid = (Np // BM,)
    F = hw.shape[1]
    Fo = Wn.shape[1]

    def body(pa_ref, pb_ref, hw_ref, dinv_ref, b_ref, w_ref, out_ref):
        d = dinv_ref[...]
        h = jnp.maximum(
            d * (pa_ref[...] + pb_ref[...] + hw_ref[...]) + b_ref[...], 0.0)
        out_ref[...] = d * jnp.dot(
            h, w_ref[...], preferred_element_type=jnp.float32)

    return pl.pallas_call(
        body,
        grid=grid,
        in_specs=[
            pl.BlockSpec((BM, F), lambda i: (i, 0)),
            pl.BlockSpec((BM, F), lambda i: (i, 0)),
            pl.BlockSpec((BM, F), lambda i: (i, 0)),
            pl.BlockSpec((BM, 1), lambda i: (i, 0)),
            pl.BlockSpec((1, F), lambda i: (0, 0)),
            pl.BlockSpec((F, Fo), lambda i: (0, 0)),
        ],
        out_specs=pl.BlockSpec((BM, Fo), lambda i: (i, 0)),
        out_shape=jax.ShapeDtypeStruct((Np, Fo), jnp.float32),
    )(pa, pb, hw, dinv, b, Wn)


def _mm_final(pa, pb, hw, dinv, b3, Wfc, bfc, Np, BM):
    grid = (Np // BM,)
    F = hw.shape[1]
    Fo = Wfc.shape[1]

    def body(pa_ref, pb_ref, hw_ref, dinv_ref, b_ref, w_ref, bfc_ref, out_ref):
        d = dinv_ref[...]
        h = jnp.maximum(
            d * (pa_ref[...] + pb_ref[...] + hw_ref[...]) + b_ref[...], 0.0)
        logits = jnp.dot(
            h, w_ref[...], preferred_element_type=jnp.float32) + bfc_ref[...]
        m = jnp.max(logits, axis=1, keepdims=True)
        lse = m + jnp.log(jnp.sum(jnp.exp(logits - m), axis=1, keepdims=True))
        out_ref[...] = logits - lse

    return pl.pallas_call(
        body,
        grid=grid,
        in_specs=[
            pl.BlockSpec((BM, F), lambda i: (i, 0)),
            pl.BlockSpec((BM, F), lambda i: (i, 0)),
            pl.BlockSpec((BM, F), lambda i: (i, 0)),
            pl.BlockSpec((BM, 1), lambda i: (i, 0)),
            pl.BlockSpec((1, F), lambda i: (0, 0)),
            pl.BlockSpec((F, Fo), lambda i: (0, 0)),
            pl.BlockSpec((1, Fo), lambda i: (0, 0)),
        ],
        out_specs=pl.BlockSpec((BM, Fo), lambda i: (i, 0)),
        out_shape=jax.ShapeDtypeStruct((Np, Fo), jnp.float32),
    )(pa, pb, hw, dinv, b3, Wfc, bfc)


def kernel(x, edge_index, edge_weight, W1, b1, W2, b2, W3, b3, Wfc, bfc):
    N = x.shape[0]
    E = edge_index.shape[1]

    Np = _round_up(N, NS * 448)
    BM = Np // 16
    Ep = _round_up(E, ROWL * CHUNK_ROWS * NW)
    Er = Ep // ROWL

    xp = jnp.pad(x, ((0, Np - N), (0, 0)))
    pe = Ep - E
    src = jnp.pad(edge_index[0], (0, pe)).reshape(Er, ROWL)
    dst = jnp.pad(edge_index[1], (0, pe), constant_values=N).reshape(Er, ROWL)
    ew = jnp.pad(edge_weight, (0, pe))

    degp = _make_deg_kernel(Np, Er)(dst, ew)
    degp = degp.reshape(NC, Np, 1)

    hw1a, hw1b, dinv = _mm_first(xp, W1, degp[0], degp[1], Np, BM)

    agg16 = _make_agg_kernel(Np, Er, 16)
    a1 = agg16(hw1a, src, dst, ew).reshape(NC, Np, 16)
    a2 = agg16(hw1b, src, dst, ew).reshape(NC, Np, 16)
    hw2 = _mm_mid2(a1[0], a1[1], a2[0], a2[1], hw1a, hw1b, dinv,
                   b1.reshape(1, -1), W2, Np, BM)

    a3 = agg16(hw2, src, dst, ew).reshape(NC, Np, 16)
    fp = 16 - W3.shape[1]
    W3p = jnp.pad(W3, ((0, 0), (0, fp)))
    b3p = jnp.pad(b3, (0, fp))
    Wfcp = jnp.pad(Wfc, ((0, fp), (0, 0)))
    hw3 = _mm_mid(a3[0], a3[1], hw2, dinv, b2.reshape(1, -1), W3p, Np, BM)

    a4 = agg16(hw3, src, dst, ew).reshape(NC, Np, 16)
    out = _mm_final(a4[0], a4[1], hw3, dinv, b3p.reshape(1, -1),
                    Wfcp, bfc.reshape(1, -1), Np, BM)

    return out[:N]


# speedup vs baseline: 28.5677x; 1.3043x over previous
import functools

import jax
import jax.numpy as jnp
from jax import lax
from jax.experimental import pallas as pl
from jax.experimental.pallas import tpu as pltpu
from jax.experimental.pallas import tpu_sc as plsc

NC = 2
NS = 16
NW = NC * NS

SPLIT0 = 0.68

ROWL = 128
CHUNK_ROWS = 16
CHUNK_E = ROWL * CHUNK_ROWS


def _round_up(a, m):
    return (a + m - 1) // m * m


def _make_deg_kernel(Np, Er):
    rows_pt = Np // NS
    er_pw = Er // NW
    n_chunks = er_pw // CHUNK_ROWS
    mesh = plsc.VectorSubcoreMesh(core_axis_name="c", subcore_axis_name="s")

    @functools.partial(
        pl.kernel,
        mesh=mesh,
        compiler_params=pltpu.CompilerParams(use_tc_tiling_on_sc=False),
        out_type=jax.ShapeDtypeStruct((NC, NS, rows_pt), jnp.float32),
        scratch_types=[
            pltpu.VMEM((CHUNK_ROWS, ROWL), jnp.int32),
            pltpu.VMEM((CHUNK_E,), jnp.float32),
            pltpu.VMEM((rows_pt,), jnp.float32),
            pltpu.VMEM_SHARED((Np,), jnp.float32),
            pltpu.SemaphoreType.DMA,
        ],
    )
    def deg_kernel(dst_hbm, ew_hbm, out_hbm, dbuf, ebuf, zbuf, acc, sem):
        cid = lax.axis_index("c")
        sid = lax.axis_index("s")
        c_pair = 2 * n_chunks
        c0 = min(int(round(c_pair * SPLIT0)), c_pair - 1)
        c1 = c_pair - c0
        my_chunks = jnp.where(cid == 0, c0, c1)

        def zero_body(i, _):
            zbuf[pl.ds(i * 16, 16)] = jnp.zeros((16,), jnp.float32)
            return 0

        lax.fori_loop(0, rows_pt // 16, zero_body, 0)
        pltpu.sync_copy(zbuf, acc.at[pl.ds(sid * rows_pt, rows_pt)])
        plsc.subcore_barrier()

        base_row = jnp.where(
            cid == 0, sid * c0 * CHUNK_ROWS,
            NS * c0 * CHUNK_ROWS + sid * c1 * CHUNK_ROWS)

        def chunk_body(c, _):
            row0 = base_row + c * CHUNK_ROWS
            pltpu.sync_copy(dst_hbm.at[pl.ds(row0, CHUNK_ROWS)], dbuf)
            pltpu.sync_copy(ew_hbm.at[pl.ds(row0 * ROWL, CHUNK_E)], ebuf)
            descs = [
                pltpu.async_copy(
                    ebuf.at[pl.ds(j * ROWL, ROWL)], acc.at[dbuf.at[j]], sem,
                    add=True)
                for j in range(CHUNK_ROWS)
            ]
            for d in descs:
                d.wait()
            return 0

        lax.fori_loop(0, my_chunks, chunk_body, 0)
        plsc.subcore_barrier()

        pltpu.sync_copy(acc.at[pl.ds(sid * rows_pt, rows_pt)], zbuf)
        pltpu.sync_copy(zbuf, out_hbm.at[cid, sid])

    return deg_kernel


def _make_agg_kernel(Np, Er, F):
    rows_pt = Np // NS
    er_pw = Er // NW
    n_chunks = er_pw // CHUNK_ROWS
    ZR = 448
    assert rows_pt % ZR == 0
    mesh = plsc.VectorSubcoreMesh(core_axis_name="c", subcore_axis_name="s")

    @functools.partial(
        pl.kernel,
        mesh=mesh,
        compiler_params=pltpu.CompilerParams(use_tc_tiling_on_sc=False),
        out_type=jax.ShapeDtypeStruct((NC, NS, rows_pt, F), jnp.float32),
        scratch_types=[
            pltpu.VMEM((CHUNK_ROWS, ROWL), jnp.int32),
            pltpu.VMEM((CHUNK_ROWS, ROWL), jnp.int32),
            pltpu.VMEM((CHUNK_E,), jnp.float32),
            pltpu.VMEM((CHUNK_E, F), jnp.float32),
            pltpu.VMEM((448, F), jnp.float32),
            pltpu.VMEM_SHARED((Np, F), jnp.float32),
            pltpu.SemaphoreType.DMA,
            pltpu.SemaphoreType.DMA,
        ],
    )
    def agg_kernel(hw_hbm, src_hbm, dst_hbm, ew_hbm, out_hbm,
                   sbuf, dbuf, ebuf, rbuf, zbuf, acc, gsem, ssem):
        cid = lax.axis_index("c")
        sid = lax.axis_index("s")
        ZR = 448
        c_pair = 2 * n_chunks
        c0 = min(int(round(c_pair * SPLIT0)), c_pair - 1)
        c1 = c_pair - c0
        my_chunks = jnp.where(cid == 0, c0, c1)

        def zero_body(i, _):
            for f0 in range(0, F, 16):
                zbuf[i, pl.ds(f0, 16)] = jnp.zeros((16,), jnp.float32)
            return 0

        lax.fori_loop(0, ZR, zero_body, 0)
        for t in range(rows_pt // ZR):
            pltpu.sync_copy(zbuf, acc.at[pl.ds(sid * rows_pt + t * ZR, ZR)])
        plsc.subcore_barrier()

        base_row = jnp.where(
            cid == 0, sid * c0 * CHUNK_ROWS,
            NS * c0 * CHUNK_ROWS + sid * c1 * CHUNK_ROWS)

        def chunk_body(c, _):
            row0 = base_row + c * CHUNK_ROWS
            pltpu.sync_copy(src_hbm.at[pl.ds(row0, CHUNK_ROWS)], sbuf)
            pltpu.sync_copy(dst_hbm.at[pl.ds(row0, CHUNK_ROWS)], dbuf)
            pltpu.sync_copy(ew_hbm.at[pl.ds(row0 * ROWL, CHUNK_E)], ebuf)
            gds = [
                pltpu.async_copy(
                    hw_hbm.at[sbuf.at[j]],
                    rbuf.at[pl.ds(j * ROWL, ROWL)], gsem)
                for j in range(CHUNK_ROWS)
            ]
            for d in gds:
                d.wait()

            def mul_body(i, _):
                e0 = i * 16
                ewv = ebuf[pl.ds(e0, 16)]
                for j in range(16):
                    e = e0 + j
                    s = jnp.full((16,), ewv[j], jnp.float32)
                    for f0 in range(0, F, 16):
                        rbuf[e, pl.ds(f0, 16)] = rbuf[e, pl.ds(f0, 16)] * s
                return 0

            lax.fori_loop(0, CHUNK_E // 16, mul_body, 0)

            sds = [
                pltpu.async_copy(
                    rbuf.at[pl.ds(j * ROWL, ROWL)],
                    acc.at[dbuf.at[j]], ssem, add=True)
                for j in range(CHUNK_ROWS)
            ]
            for d in sds:
                d.wait()
            return 0

        lax.fori_loop(0, my_chunks, chunk_body, 0)
        plsc.subcore_barrier()

        for t in range(rows_pt // ZR):
            pltpu.sync_copy(acc.at[pl.ds(sid * rows_pt + t * ZR, ZR)], zbuf)
            pltpu.sync_copy(zbuf, out_hbm.at[cid, sid, pl.ds(t * ZR, ZR)])

    return agg_kernel


def _mm_first(x, W1, degA, degB, Np, BM):
    grid = (Np // BM,)
    kd = x.shape[1]
    Fo = W1.shape[1]
    Fh = Fo // 2

    def body(x_ref, w_ref, da_ref, db_ref, hwa_ref, hwb_ref, dinv_ref):
        dinv = 1.0 / jnp.sqrt(1.0 + da_ref[...] + db_ref[...])
        hw = dinv * jnp.dot(
            x_ref[...], w_ref[...], preferred_element_type=jnp.float32)
        hwa_ref[...] = hw[:, :Fh]
        hwb_ref[...] = hw[:, Fh:]
        dinv_ref[...] = dinv

    return pl.pallas_call(
        body,
        grid=grid,
        in_specs=[
            pl.BlockSpec((BM, kd), lambda i: (i, 0)),
            pl.BlockSpec((kd, Fo), lambda i: (0, 0)),
            pl.BlockSpec((BM, 1), lambda i: (i, 0)),
            pl.BlockSpec((BM, 1), lambda i: (i, 0)),
        ],
        out_specs=[
            pl.BlockSpec((BM, Fh), lambda i: (i, 0)),
            pl.BlockSpec((BM, Fh), lambda i: (i, 0)),
            pl.BlockSpec((BM, 1), lambda i: (i, 0)),
        ],
        out_shape=[
            jax.ShapeDtypeStruct((Np, Fh), jnp.float32),
            jax.ShapeDtypeStruct((Np, Fh), jnp.float32),
            jax.ShapeDtypeStruct((Np, 1), jnp.float32),
        ],
    )(x, W1, degA, degB)


def _agg_spec(BM, F):
    return [pl.BlockSpec((1, 1, BM, F), lambda i: (0, i, 0, 0)),
            pl.BlockSpec((1, 1, BM, F), lambda i: (1, i, 0, 0))]


def _mm_mid2(agg1, agg2, hwa, hwb, dinv, b, Wn, Np, BM):
    grid = (Np // BM,)
    Fh = hwa.shape[1]
    Fo = Wn.shape[1]

    def body(pa1_ref, pb1_ref, pa2_ref, pb2_ref, hwa_ref, hwb_ref,
             dinv_ref, b_ref, w_ref, out_ref):
        d = dinv_ref[...]
        agg = jnp.concatenate(
            [(pa1_ref[...] + pb1_ref[...]).reshape(BM, Fh) + hwa_ref[...],
             (pa2_ref[...] + pb2_ref[...]).reshape(BM, Fh) + hwb_ref[...]],
            axis=1)
        h = jnp.maximum(d * agg + b_ref[...], 0.0)
        out_ref[...] = d * jnp.dot(
            h, w_ref[...], preferred_element_type=jnp.float32)

    return pl.pallas_call(
        body,
        grid=grid,
        in_specs=_agg_spec(BM, Fh) + _agg_spec(BM, Fh) + [
            pl.BlockSpec((BM, Fh), lambda i: (i, 0)),
            pl.BlockSpec((BM, Fh), lambda i: (i, 0)),
            pl.BlockSpec((BM, 1), lambda i: (i, 0)),
            pl.BlockSpec((1, 2 * Fh), lambda i: (0, 0)),
            pl.BlockSpec((2 * Fh, Fo), lambda i: (0, 0)),
        ],
        out_specs=pl.BlockSpec((BM, Fo), lambda i: (i, 0)),
        out_shape=jax.ShapeDtypeStruct((Np, Fo), jnp.float32),
    )(agg1, agg1, agg2, agg2, hwa, hwb, dinv, b, Wn)


def _mm_mid(aggp, hw, dinv, b, Wn, Np, BM):
    grid = (Np // BM,)
    F = hw.shape[1]
    Fo = Wn.shape[1]

    def body(pa_ref, pb_ref, hw_ref, dinv_ref, b_ref, w_ref, out_ref):
        d = dinv_ref[...]
        agg = (pa_ref[...] + pb_ref[...]).reshape(BM, F) + hw_ref[...]
        h = jnp.maximum(d * agg + b_ref[...], 0.0)
        out_ref[...] = d * jnp.dot(
            h, w_ref[...], preferred_element_type=jnp.float32)

    return pl.pallas_call(
        body,
        grid=grid,
        in_specs=_agg_spec(BM, F) + [
            pl.BlockSpec((BM, F), lambda i: (i, 0)),
            pl.BlockSpec((BM, 1), lambda i: (i, 0)),
            pl.BlockSpec((1, F), lambda i: (0, 0)),
            pl.BlockSpec((F, Fo), lambda i: (0, 0)),
        ],
        out_specs=pl.BlockSpec((BM, Fo), lambda i: (i, 0)),
        out_shape=jax.ShapeDtypeStruct((Np, Fo), jnp.float32),
    )(aggp, aggp, hw, dinv, b, Wn)


def _mm_final(aggp, hw, dinv, b3, Wfc, bfc, Np, BM):
    grid = (Np // BM,)
    F = hw.shape[1]
    Fo = Wfc.shape[1]

    def body(pa_ref, pb_ref, hw_ref, dinv_ref, b_ref, w_ref, bfc_ref, out_ref):
        d = dinv_ref[...]
        agg = (pa_ref[...] + pb_ref[...]).reshape(BM, F) + hw_ref[...]
        h = jnp.maximum(d * agg + b_ref[...], 0.0)
        logits = jnp.dot(
            h, w_ref[...], preferred_element_type=jnp.float32) + bfc_ref[...]
        m = jnp.max(logits, axis=1, keepdims=True)
        lse = m + jnp.log(jnp.sum(jnp.exp(logits - m), axis=1, keepdims=True))
        out_ref[...] = logits - lse

    return pl.pallas_call(
        body,
        grid=grid,
        in_specs=_agg_spec(BM, F) + [
            pl.BlockSpec((BM, F), lambda i: (i, 0)),
            pl.BlockSpec((BM, 1), lambda i: (i, 0)),
            pl.BlockSpec((1, F), lambda i: (0, 0)),
            pl.BlockSpec((F, Fo), lambda i: (0, 0)),
            pl.BlockSpec((1, Fo), lambda i: (0, 0)),
        ],
        out_specs=pl.BlockSpec((BM, Fo), lambda i: (i, 0)),
        out_shape=jax.ShapeDtypeStruct((Np, Fo), jnp.float32),
    )(aggp, aggp, hw, dinv, b3, Wfc, bfc)


def kernel(x, edge_index, edge_weight, W1, b1, W2, b2, W3, b3, Wfc, bfc):
    N = x.shape[0]
    E = edge_index.shape[1]

    Np = _round_up(N, NS * 448)
    BM = Np // 16
    Ep = _round_up(E, ROWL * CHUNK_ROWS * NW)
    Er = Ep // ROWL

    xp = jnp.pad(x, ((0, Np - N), (0, 0)))
    pe = Ep - E
    src = jnp.pad(edge_index[0], (0, pe)).reshape(Er, ROWL)
    dst = jnp.pad(edge_index[1], (0, pe), constant_values=N).reshape(Er, ROWL)
    ew = jnp.pad(edge_weight, (0, pe))

    degp = _make_deg_kernel(Np, Er)(dst, ew)
    degp = degp.reshape(NC, Np, 1)

    hw1a, hw1b, dinv = _mm_first(xp, W1, degp[0], degp[1], Np, BM)

    agg16 = _make_agg_kernel(Np, Er, 16)
    a1 = agg16(hw1a, src, dst, ew)
    a2 = agg16(hw1b, src, dst, ew)
    hw2 = _mm_mid2(a1, a2, hw1a, hw1b, dinv,
                   b1.reshape(1, -1), W2, Np, BM)

    a3 = agg16(hw2, src, dst, ew)
    fp = 16 - W3.shape[1]
    W3p = jnp.pad(W3, ((0, 0), (0, fp)))
    b3p = jnp.pad(b3, (0, fp))
    Wfcp = jnp.pad(Wfc, ((0, fp), (0, 0)))
    hw3 = _mm_mid(a3, hw2, dinv, b2.reshape(1, -1), W3p, Np, BM)

    a4 = agg16(hw3, src, dst, ew)
    out = _mm_final(a4, hw3, dinv, b3p.reshape(1, -1),
                    Wfcp, bfc.reshape(1, -1), Np, BM)

    return out[:N]


# speedup vs baseline: 29.6876x; 1.0392x over previous
import functools

import jax
import jax.numpy as jnp
from jax import lax
from jax.experimental import pallas as pl
from jax.experimental.pallas import tpu as pltpu
from jax.experimental.pallas import tpu_sc as plsc

NC = 2
NS = 16
NW = NC * NS

SPLIT0 = 0.72

ROWL = 128
CHUNK_ROWS = 16
CHUNK_E = ROWL * CHUNK_ROWS


def _round_up(a, m):
    return (a + m - 1) // m * m


def _make_deg_kernel(Np, Er):
    rows_pt = Np // NS
    er_pw = Er // NW
    n_chunks = er_pw // CHUNK_ROWS
    mesh = plsc.VectorSubcoreMesh(core_axis_name="c", subcore_axis_name="s")

    @functools.partial(
        pl.kernel,
        mesh=mesh,
        compiler_params=pltpu.CompilerParams(use_tc_tiling_on_sc=False),
        out_type=jax.ShapeDtypeStruct((NC, NS, rows_pt), jnp.float32),
        scratch_types=[
            pltpu.VMEM((CHUNK_ROWS, ROWL), jnp.int32),
            pltpu.VMEM((CHUNK_E,), jnp.float32),
            pltpu.VMEM((rows_pt,), jnp.float32),
            pltpu.VMEM_SHARED((Np,), jnp.float32),
            pltpu.SemaphoreType.DMA,
        ],
    )
    def deg_kernel(dst_hbm, ew_hbm, out_hbm, dbuf, ebuf, zbuf, acc, sem):
        cid = lax.axis_index("c")
        sid = lax.axis_index("s")
        c_pair = Er // (CHUNK_ROWS * NS)
        c0 = min(int(round(c_pair * SPLIT0)), c_pair - 1)
        c1 = c_pair - c0
        my_chunks = jnp.where(cid == 0, c0, c1)

        def zero_body(i, _):
            zbuf[pl.ds(i * 16, 16)] = jnp.zeros((16,), jnp.float32)
            return 0

        lax.fori_loop(0, rows_pt // 16, zero_body, 0)
        pltpu.sync_copy(zbuf, acc.at[pl.ds(sid * rows_pt, rows_pt)])
        plsc.subcore_barrier()

        base_row = jnp.where(
            cid == 0, sid * c0 * CHUNK_ROWS,
            NS * c0 * CHUNK_ROWS + sid * c1 * CHUNK_ROWS)

        def chunk_body(c, _):
            row0 = base_row + c * CHUNK_ROWS
            pltpu.sync_copy(dst_hbm.at[pl.ds(row0, CHUNK_ROWS)], dbuf)
            pltpu.sync_copy(ew_hbm.at[pl.ds(row0 * ROWL, CHUNK_E)], ebuf)
            descs = [
                pltpu.async_copy(
                    ebuf.at[pl.ds(j * ROWL, ROWL)], acc.at[dbuf.at[j]], sem,
                    add=True)
                for j in range(CHUNK_ROWS)
            ]
            for d in descs:
                d.wait()
            return 0

        lax.fori_loop(0, my_chunks, chunk_body, 0)
        plsc.subcore_barrier()

        pltpu.sync_copy(acc.at[pl.ds(sid * rows_pt, rows_pt)], zbuf)
        pltpu.sync_copy(zbuf, out_hbm.at[cid, sid])

    return deg_kernel


def _make_agg_kernel(Np, Er, F):
    rows_pt = Np // NS
    er_pw = Er // NW
    n_chunks = er_pw // CHUNK_ROWS
    ZR = 448
    assert rows_pt % ZR == 0
    mesh = plsc.VectorSubcoreMesh(core_axis_name="c", subcore_axis_name="s")

    @functools.partial(
        pl.kernel,
        mesh=mesh,
        compiler_params=pltpu.CompilerParams(use_tc_tiling_on_sc=False),
        out_type=jax.ShapeDtypeStruct((NC, NS, rows_pt, F), jnp.float32),
        scratch_types=[
            pltpu.VMEM((CHUNK_ROWS, ROWL), jnp.int32),
            pltpu.VMEM((CHUNK_ROWS, ROWL), jnp.int32),
            pltpu.VMEM((CHUNK_E,), jnp.float32),
            pltpu.VMEM((CHUNK_E, F), jnp.float32),
            pltpu.VMEM((448, F), jnp.float32),
            pltpu.VMEM_SHARED((Np, F), jnp.float32),
            pltpu.SemaphoreType.DMA,
            pltpu.SemaphoreType.DMA,
        ],
    )
    def agg_kernel(hw_hbm, src_hbm, dst_hbm, ew_hbm, out_hbm,
                   sbuf, dbuf, ebuf, rbuf, zbuf, acc, gsem, ssem):
        cid = lax.axis_index("c")
        sid = lax.axis_index("s")
        ZR = 448
        c_pair = Er // (CHUNK_ROWS * NS)
        c0 = min(int(round(c_pair * SPLIT0)), c_pair - 1)
        c1 = c_pair - c0
        my_chunks = jnp.where(cid == 0, c0, c1)

        def zero_body(i, _):
            for f0 in range(0, F, 16):
                zbuf[i, pl.ds(f0, 16)] = jnp.zeros((16,), jnp.float32)
            return 0

        lax.fori_loop(0, ZR, zero_body, 0)
        for t in range(rows_pt // ZR):
            pltpu.sync_copy(zbuf, acc.at[pl.ds(sid * rows_pt + t * ZR, ZR)])
        plsc.subcore_barrier()

        base_row = jnp.where(
            cid == 0, sid * c0 * CHUNK_ROWS,
            NS * c0 * CHUNK_ROWS + sid * c1 * CHUNK_ROWS)

        def chunk_body(c, _):
            row0 = base_row + c * CHUNK_ROWS
            pltpu.sync_copy(src_hbm.at[pl.ds(row0, CHUNK_ROWS)], sbuf)
            pltpu.sync_copy(dst_hbm.at[pl.ds(row0, CHUNK_ROWS)], dbuf)
            pltpu.sync_copy(ew_hbm.at[pl.ds(row0 * ROWL, CHUNK_E)], ebuf)
            gds = [
                pltpu.async_copy(
                    hw_hbm.at[sbuf.at[j]],
                    rbuf.at[pl.ds(j * ROWL, ROWL)], gsem)
                for j in range(CHUNK_ROWS)
            ]
            for d in gds:
                d.wait()

            def mul_body(i, _):
                e0 = i * 16
                ewv = ebuf[pl.ds(e0, 16)]
                for j in range(16):
                    e = e0 + j
                    s = jnp.full((16,), ewv[j], jnp.float32)
                    for f0 in range(0, F, 16):
                        rbuf[e, pl.ds(f0, 16)] = rbuf[e, pl.ds(f0, 16)] * s
                return 0

            lax.fori_loop(0, CHUNK_E // 16, mul_body, 0)

            sds = [
                pltpu.async_copy(
                    rbuf.at[pl.ds(j * ROWL, ROWL)],
                    acc.at[dbuf.at[j]], ssem, add=True)
                for j in range(CHUNK_ROWS)
            ]
            for d in sds:
                d.wait()
            return 0

        lax.fori_loop(0, my_chunks, chunk_body, 0)
        plsc.subcore_barrier()

        for t in range(rows_pt // ZR):
            pltpu.sync_copy(acc.at[pl.ds(sid * rows_pt + t * ZR, ZR)], zbuf)
            pltpu.sync_copy(zbuf, out_hbm.at[cid, sid, pl.ds(t * ZR, ZR)])

    return agg_kernel


def _mm_first(x, W1, degA, degB, Np, BM):
    grid = (Np // BM,)
    kd = x.shape[1]
    Fo = W1.shape[1]
    Fh = Fo // 2

    def body(x_ref, w_ref, da_ref, db_ref, hwa_ref, hwb_ref, dinv_ref):
        dinv = 1.0 / jnp.sqrt(1.0 + da_ref[...] + db_ref[...])
        hw = dinv * jnp.dot(
            x_ref[...], w_ref[...], preferred_element_type=jnp.float32)
        hwa_ref[...] = hw[:, :Fh]
        hwb_ref[...] = hw[:, Fh:]
        dinv_ref[...] = dinv

    return pl.pallas_call(
        body,
        grid=grid,
        in_specs=[
            pl.BlockSpec((BM, kd), lambda i: (i, 0)),
            pl.BlockSpec((kd, Fo), lambda i: (0, 0)),
            pl.BlockSpec((BM, 1), lambda i: (i, 0)),
            pl.BlockSpec((BM, 1), lambda i: (i, 0)),
        ],
        out_specs=[
            pl.BlockSpec((BM, Fh), lambda i: (i, 0)),
            pl.BlockSpec((BM, Fh), lambda i: (i, 0)),
            pl.BlockSpec((BM, 1), lambda i: (i, 0)),
        ],
        out_shape=[
            jax.ShapeDtypeStruct((Np, Fh), jnp.float32),
            jax.ShapeDtypeStruct((Np, Fh), jnp.float32),
            jax.ShapeDtypeStruct((Np, 1), jnp.float32),
        ],
    )(x, W1, degA, degB)


def _agg_spec(BM, F):
    return [pl.BlockSpec((1, 1, BM, F), lambda i: (0, i, 0, 0)),
            pl.BlockSpec((1, 1, BM, F), lambda i: (1, i, 0, 0))]


def _mm_mid2(agg1, agg2, hwa, hwb, dinv, b, Wn, Np, BM):
    grid = (Np // BM,)
    Fh = hwa.shape[1]
    Fo = Wn.shape[1]

    def body(pa1_ref, pb1_ref, pa2_ref, pb2_ref, hwa_ref, hwb_ref,
             dinv_ref, b_ref, w_ref, out_ref):
        d = dinv_ref[...]
        agg = jnp.concatenate(
            [(pa1_ref[...] + pb1_ref[...]).reshape(BM, Fh) + hwa_ref[...],
             (pa2_ref[...] + pb2_ref[...]).reshape(BM, Fh) + hwb_ref[...]],
            axis=1)
        h = jnp.maximum(d * agg + b_ref[...], 0.0)
        out_ref[...] = d * jnp.dot(
            h, w_ref[...], preferred_element_type=jnp.float32)

    return pl.pallas_call(
        body,
        grid=grid,
        in_specs=_agg_spec(BM, Fh) + _agg_spec(BM, Fh) + [
            pl.BlockSpec((BM, Fh), lambda i: (i, 0)),
            pl.BlockSpec((BM, Fh), lambda i: (i, 0)),
            pl.BlockSpec((BM, 1), lambda i: (i, 0)),
            pl.BlockSpec((1, 2 * Fh), lambda i: (0, 0)),
            pl.BlockSpec((2 * Fh, Fo), lambda i: (0, 0)),
        ],
        out_specs=pl.BlockSpec((BM, Fo), lambda i: (i, 0)),
        out_shape=jax.ShapeDtypeStruct((Np, Fo), jnp.float32),
    )(agg1, agg1, agg2, agg2, hwa, hwb, dinv, b, Wn)


def _mm_mid(aggp, hw, dinv, b, Wn, Np, BM):
    grid = (Np // BM,)
    F = hw.shape[1]
    Fo = Wn.shape[1]

    def body(pa_ref, pb_ref, hw_ref, dinv_ref, b_ref, w_ref, out_ref):
        d = dinv_ref[...]
        agg = (pa_ref[...] + pb_ref[...]).reshape(BM, F) + hw_ref[...]
        h = jnp.maximum(d * agg + b_ref[...], 0.0)
        out_ref[...] = d * jnp.dot(
            h, w_ref[...], preferred_element_type=jnp.float32)

    return pl.pallas_call(
        body,
        grid=grid,
        in_specs=_agg_spec(BM, F) + [
            pl.BlockSpec((BM, F), lambda i: (i, 0)),
            pl.BlockSpec((BM, 1), lambda i: (i, 0)),
            pl.BlockSpec((1, F), lambda i: (0, 0)),
            pl.BlockSpec((F, Fo), lambda i: (0, 0)),
        ],
        out_specs=pl.BlockSpec((BM, Fo), lambda i: (i, 0)),
        out_shape=jax.ShapeDtypeStruct((Np, Fo), jnp.float32),
    )(aggp, aggp, hw, dinv, b, Wn)


def _mm_final(aggp, hw, dinv, b3, Wfc, bfc, Np, BM):
    grid = (Np // BM,)
    F = hw.shape[1]
    Fo = Wfc.shape[1]

    def body(pa_ref, pb_ref, hw_ref, dinv_ref, b_ref, w_ref, bfc_ref, out_ref):
        d = dinv_ref[...]
        agg = (pa_ref[...] + pb_ref[...]).reshape(BM, F) + hw_ref[...]
        h = jnp.maximum(d * agg + b_ref[...], 0.0)
        logits = jnp.dot(
            h, w_ref[...], preferred_element_type=jnp.float32) + bfc_ref[...]
        m = jnp.max(logits, axis=1, keepdims=True)
        lse = m + jnp.log(jnp.sum(jnp.exp(logits - m), axis=1, keepdims=True))
        out_ref[...] = logits - lse

    return pl.pallas_call(
        body,
        grid=grid,
        in_specs=_agg_spec(BM, F) + [
            pl.BlockSpec((BM, F), lambda i: (i, 0)),
            pl.BlockSpec((BM, 1), lambda i: (i, 0)),
            pl.BlockSpec((1, F), lambda i: (0, 0)),
            pl.BlockSpec((F, Fo), lambda i: (0, 0)),
            pl.BlockSpec((1, Fo), lambda i: (0, 0)),
        ],
        out_specs=pl.BlockSpec((BM, Fo), lambda i: (i, 0)),
        out_shape=jax.ShapeDtypeStruct((Np, Fo), jnp.float32),
    )(aggp, aggp, hw, dinv, b3, Wfc, bfc)


def kernel(x, edge_index, edge_weight, W1, b1, W2, b2, W3, b3, Wfc, bfc):
    N = x.shape[0]
    E = edge_index.shape[1]

    Np = _round_up(N, NS * 448)
    BM = Np // 16
    Ep = _round_up(E, ROWL * CHUNK_ROWS * NW)
    Er = Ep // ROWL

    xp = jnp.pad(x, ((0, Np - N), (0, 0)))
    pe = Ep - E
    src = jnp.pad(edge_index[0], (0, pe)).reshape(Er, ROWL)
    dst = jnp.pad(edge_index[1], (0, pe), constant_values=N).reshape(Er, ROWL)
    ew = jnp.pad(edge_weight, (0, pe))

    degp = _make_deg_kernel(Np, Er)(dst, ew)
    degp = degp.reshape(NC, Np, 1)

    hw1a, hw1b, dinv = _mm_first(xp, W1, degp[0], degp[1], Np, BM)

    agg16 = _make_agg_kernel(Np, Er, 16)
    a1 = agg16(hw1a, src, dst, ew)
    a2 = agg16(hw1b, src, dst, ew)
    hw2 = _mm_mid2(a1, a2, hw1a, hw1b, dinv,
                   b1.reshape(1, -1), W2, Np, BM)

    a3 = agg16(hw2, src, dst, ew)
    fp = 16 - W3.shape[1]
    W3p = jnp.pad(W3, ((0, 0), (0, fp)))
    b3p = jnp.pad(b3, (0, fp))
    Wfcp = jnp.pad(Wfc, ((0, fp), (0, 0)))
    hw3 = _mm_mid(a3, hw2, dinv, b2.reshape(1, -1), W3p, Np, BM)

    a4 = agg16(hw3, src, dst, ew)
    out = _mm_final(a4, hw3, dinv, b3p.reshape(1, -1),
                    Wfcp, bfc.reshape(1, -1), Np, BM)

    return out[:N]
